# baseline (device time: 18441 ns/iter reference)
import jax
import jax.numpy as jnp
from jax import lax
from jax.experimental import pallas as pl
from jax.experimental.pallas import tpu as pltpu

M = 1024
HALF = 512
ROWS = 512
CROWS = 32
NSELF = ROWS // CROWS
EXTRA = 4
NPULL = NSELF + EXTRA
NFWD = NSELF - EXTRA


def kernel(x):
    def body(
        x_ref,
        out_ref,
        in_all,
        send_y,
        recv_y,
        sum_buf,
        dsem,
        osem,
        ysend,
        yrecv,
        zsend,
        zrecv,
    ):
        my_x = lax.axis_index("x")
        my_y = lax.axis_index("y")
        my_z = lax.axis_index("z")
        ypeer = (my_x, 1 - my_y, my_z)
        znb = (my_x, my_y, 1 - my_z)

        row0 = my_z * ROWS
        other0 = (1 - my_z) * ROWS
        my_col = my_y * HALF
        peer_col = (1 - my_y) * HALF

        def chunk_row(c):
            if c < NSELF:
                return row0 + c * CROWS
            return other0 + (NSELF - EXTRA + (c - NSELF)) * CROWS

        barrier = pltpu.get_barrier_semaphore()
        for nbr in (ypeer, znb):
            pl.semaphore_signal(
                barrier, inc=1, device_id=nbr, device_id_type=pl.DeviceIdType.MESH
            )

        dmas = []
        for c in range(NPULL):
            rows = pl.ds(chunk_row(c), CROWS)
            dm = pltpu.make_async_copy(x_ref.at[0, rows, :], in_all.at[c], dsem.at[c])
            dm.start()
            dmas.append(dm)

        pl.semaphore_wait(barrier, 2)

        rdmas_y = []
        for c in range(NPULL):
            dmas[c].wait()
            send_y[c] = in_all[c, :, pl.ds(peer_col, HALF)].astype(jnp.bfloat16)
            ry = pltpu.make_async_remote_copy(
                src_ref=send_y.at[c],
                dst_ref=recv_y.at[c],
                send_sem=ysend.at[c],
                recv_sem=yrecv.at[c],
                device_id=ypeer,
                device_id_type=pl.DeviceIdType.MESH,
            )
            ry.start()
            rdmas_y.append(ry)

        rdmas_z = []
        copies = []
        for c in range(NPULL):
            rdmas_y[c].wait()
            rows = pl.ds(chunk_row(c), CROWS)
            sum_buf[c] = (
                in_all[c, :, pl.ds(my_col, HALF)].astype(jnp.bfloat16) + recv_y[c]
            )
            if c < NFWD:
                rz = pltpu.make_async_remote_copy(
                    src_ref=sum_buf.at[c],
                    dst_ref=out_ref.at[rows],
                    send_sem=zsend.at[c],
                    recv_sem=zrecv.at[c],
                    device_id=znb,
                    device_id_type=pl.DeviceIdType.MESH,
                )
                rz.start()
                rdmas_z.append(rz)
            cp = pltpu.make_async_copy(sum_buf.at[c], out_ref.at[rows], osem.at[c])
            cp.start()
            copies.append(cp)

        for cp in copies:
            cp.wait()
        for rz in rdmas_z:
            rz.wait()

    return pl.pallas_call(
        body,
        out_shape=jax.ShapeDtypeStruct((M, HALF), jnp.bfloat16),
        in_specs=[pl.BlockSpec(memory_space=pl.ANY)],
        out_specs=pl.BlockSpec(memory_space=pl.ANY),
        scratch_shapes=[
            pltpu.VMEM((NPULL, CROWS, 2 * HALF), jnp.float32),
            pltpu.VMEM((NPULL, CROWS, HALF), jnp.bfloat16),
            pltpu.VMEM((NPULL, CROWS, HALF), jnp.bfloat16),
            pltpu.VMEM((NPULL, CROWS, HALF), jnp.bfloat16),
            pltpu.SemaphoreType.DMA((NPULL,)),
            pltpu.SemaphoreType.DMA((NPULL,)),
            pltpu.SemaphoreType.DMA((NPULL,)),
            pltpu.SemaphoreType.DMA((NPULL,)),
            pltpu.SemaphoreType.DMA((NFWD,)),
            pltpu.SemaphoreType.DMA((NFWD,)),
        ],
        compiler_params=pltpu.CompilerParams(collective_id=0),
    )(x)


# device time: 15717 ns/iter; 1.1733x vs baseline; 1.1733x over previous
import jax
import jax.numpy as jnp
from jax import lax
from jax.experimental import pallas as pl
from jax.experimental.pallas import tpu as pltpu

M = 1024
HALF = 512
ROWS = 512
CROWS = 128
NSELF = ROWS // CROWS
EXTRA = 1
NPULL = NSELF + EXTRA
NFWD = NSELF - EXTRA


def kernel(x):
    def body(
        x_ref,
        out_ref,
        in_all,
        send_y,
        recv_y,
        sum_buf,
        dsem,
        osem,
        ysend,
        yrecv,
        zsend,
        zrecv,
    ):
        my_x = lax.axis_index("x")
        my_y = lax.axis_index("y")
        my_z = lax.axis_index("z")
        ypeer = (my_x, 1 - my_y, my_z)
        znb = (my_x, my_y, 1 - my_z)

        row0 = my_z * ROWS
        other0 = (1 - my_z) * ROWS
        my_col = my_y * HALF
        peer_col = (1 - my_y) * HALF

        def chunk_row(c):
            if c < NSELF:
                return row0 + c * CROWS
            return other0 + (NSELF - EXTRA + (c - NSELF)) * CROWS

        barrier = pltpu.get_barrier_semaphore()
        for nbr in (ypeer, znb):
            pl.semaphore_signal(
                barrier, inc=1, device_id=nbr, device_id_type=pl.DeviceIdType.MESH
            )

        dmas = []
        for c in range(NPULL):
            rows = pl.ds(chunk_row(c), CROWS)
            dm = pltpu.make_async_copy(x_ref.at[0, rows, :], in_all.at[c], dsem.at[c])
            dm.start()
            dmas.append(dm)

        dmas[0].wait()
        send_y[0] = in_all[0, :, pl.ds(peer_col, HALF)].astype(jnp.bfloat16)
        pl.semaphore_wait(barrier, 2)

        rdmas_y = []
        for c in range(NPULL):
            if c > 0:
                dmas[c].wait()
                send_y[c] = in_all[c, :, pl.ds(peer_col, HALF)].astype(jnp.bfloat16)
            ry = pltpu.make_async_remote_copy(
                src_ref=send_y.at[c],
                dst_ref=recv_y.at[c],
                send_sem=ysend.at[c],
                recv_sem=yrecv.at[c],
                device_id=ypeer,
                device_id_type=pl.DeviceIdType.MESH,
            )
            ry.start()
            rdmas_y.append(ry)

        rdmas_z = []
        copies = []
        for c in range(NPULL):
            rdmas_y[c].wait()
            rows = pl.ds(chunk_row(c), CROWS)
            sum_buf[c] = (
                in_all[c, :, pl.ds(my_col, HALF)].astype(jnp.bfloat16) + recv_y[c]
            )
            if c < NFWD:
                rz = pltpu.make_async_remote_copy(
                    src_ref=sum_buf.at[c],
                    dst_ref=out_ref.at[rows],
                    send_sem=zsend.at[c],
                    recv_sem=zrecv.at[c],
                    device_id=znb,
                    device_id_type=pl.DeviceIdType.MESH,
                )
                rz.start()
                rdmas_z.append(rz)
            cp = pltpu.make_async_copy(sum_buf.at[c], out_ref.at[rows], osem.at[c])
            cp.start()
            copies.append(cp)

        for cp in copies:
            cp.wait()
        for rz in rdmas_z:
            rz.wait()

    return pl.pallas_call(
        body,
        out_shape=jax.ShapeDtypeStruct((M, HALF), jnp.bfloat16),
        in_specs=[pl.BlockSpec(memory_space=pl.ANY)],
        out_specs=pl.BlockSpec(memory_space=pl.ANY),
        scratch_shapes=[
            pltpu.VMEM((NPULL, CROWS, 2 * HALF), jnp.float32),
            pltpu.VMEM((NPULL, CROWS, HALF), jnp.bfloat16),
            pltpu.VMEM((NPULL, CROWS, HALF), jnp.bfloat16),
            pltpu.VMEM((NPULL, CROWS, HALF), jnp.bfloat16),
            pltpu.SemaphoreType.DMA((NPULL,)),
            pltpu.SemaphoreType.DMA((NPULL,)),
            pltpu.SemaphoreType.DMA((NPULL,)),
            pltpu.SemaphoreType.DMA((NPULL,)),
            pltpu.SemaphoreType.DMA((NFWD,)),
            pltpu.SemaphoreType.DMA((NFWD,)),
        ],
        compiler_params=pltpu.CompilerParams(collective_id=0),
    )(x)
